# Initial kernel scaffold; baseline (speedup 1.0000x reference)
import jax
import jax.numpy as jnp
from jax.experimental import pallas as pl


def kernel(src_table, tgt_table, src_tokens, tgt_tokens):
    raise NotImplementedError("write your pallas kernel here")


# speedup vs baseline: 1.0979x; 1.0979x over previous
import functools

import jax
import jax.numpy as jnp
from jax import lax
from jax.experimental import pallas as pl
from jax.experimental.pallas import tpu as pltpu
from jax.experimental.pallas import tpu_sc as plsc

_EMBED = 128
_CHUNK = 128
_NSUB = 16


def _gather_side(table, idx_hbm, out_hbm, idx_v, buf, sem, sid, n_chunks):
    rows_per_worker = n_chunks * _CHUNK
    base = sid * rows_per_worker
    pltpu.sync_copy(idx_hbm.at[sid], idx_v)

    def body(j, carry):
        pltpu.async_copy(table.at[idx_v.at[j]], buf, sem).wait()
        pltpu.sync_copy(buf, out_hbm.at[pl.ds(base + j * _CHUNK, _CHUNK)])
        return carry

    lax.fori_loop(0, n_chunks, body, 0)


@functools.lru_cache(maxsize=None)
def _build(n_rows):
    assert n_rows % (_NSUB * _CHUNK) == 0
    n_chunks = n_rows // (_NSUB * _CHUNK)
    mesh = plsc.VectorSubcoreMesh(core_axis_name="c", subcore_axis_name="s")

    @functools.partial(
        pl.kernel,
        out_type=[
            jax.ShapeDtypeStruct((n_rows, _EMBED), jnp.float32),
            jax.ShapeDtypeStruct((n_rows, _EMBED), jnp.float32),
        ],
        scratch_types=[
            pltpu.VMEM((n_chunks, _CHUNK), jnp.int32),
            pltpu.VMEM((_CHUNK, _EMBED), jnp.float32),
            pltpu.SemaphoreType.DMA,
        ],
        mesh=mesh,
    )
    def emb_kernel(src_table, tgt_table, src_idx, tgt_idx,
                   src_out, tgt_out, idx_v, buf, sem):
        cid = lax.axis_index("c")
        sid = lax.axis_index("s")

        @pl.when(cid == 0)
        def _():
            _gather_side(src_table, src_idx, src_out, idx_v, buf, sem,
                         sid, n_chunks)

        @pl.when(cid == 1)
        def _():
            _gather_side(tgt_table, tgt_idx, tgt_out, idx_v, buf, sem,
                         sid, n_chunks)

    return emb_kernel


def kernel(src_table, tgt_table, src_tokens, tgt_tokens):
    b, s = src_tokens.shape
    n_rows = b * s
    n_chunks = n_rows // (_NSUB * _CHUNK)
    src_idx = src_tokens.astype(jnp.int32).reshape(_NSUB, n_chunks, _CHUNK)
    tgt_idx = tgt_tokens.astype(jnp.int32).reshape(_NSUB, n_chunks, _CHUNK)
    src_out, tgt_out = _build(n_rows)(src_table, tgt_table, src_idx, tgt_idx)
    return (
        src_out.reshape(b, s, _EMBED),
        tgt_out.reshape(b, s, _EMBED),
    )


# speedup vs baseline: 1.1513x; 1.0487x over previous
import functools

import jax
import jax.numpy as jnp
from jax import lax
from jax.experimental import pallas as pl
from jax.experimental.pallas import tpu as pltpu
from jax.experimental.pallas import tpu_sc as plsc

_EMBED = 128
_CHUNK = 64
_NSUB = 16


def _gather_side(table, idx_hbm, out_hbm, idx_v, buf0, buf1, gs0, gs1,
                 sid, n_chunks):
    rows_per_worker = n_chunks * _CHUNK
    base = sid * rows_per_worker
    pltpu.sync_copy(idx_hbm.at[sid], idx_v)

    pltpu.async_copy(table.at[idx_v.at[0]], buf0, gs0)
    pltpu.async_copy(table.at[idx_v.at[1]], buf1, gs1)

    def body(i, carry):
        j = 2 * i

        pltpu.make_async_copy(table.at[idx_v.at[j]], buf0, gs0).wait()
        pltpu.sync_copy(buf0, out_hbm.at[pl.ds(base + j * _CHUNK, _CHUNK)])

        @pl.when(j + 2 < n_chunks)
        def _():
            pltpu.async_copy(table.at[idx_v.at[j + 2]], buf0, gs0)

        pltpu.make_async_copy(table.at[idx_v.at[j + 1]], buf1, gs1).wait()
        pltpu.sync_copy(
            buf1, out_hbm.at[pl.ds(base + (j + 1) * _CHUNK, _CHUNK)])

        @pl.when(j + 3 < n_chunks)
        def _():
            pltpu.async_copy(table.at[idx_v.at[j + 3]], buf1, gs1)

        return carry

    lax.fori_loop(0, n_chunks // 2, body, 0)


@functools.lru_cache(maxsize=None)
def _build(n_rows):
    assert n_rows % (_NSUB * _CHUNK * 2) == 0
    n_chunks = n_rows // (_NSUB * _CHUNK)
    mesh = plsc.VectorSubcoreMesh(core_axis_name="c", subcore_axis_name="s")

    @functools.partial(
        pl.kernel,
        out_type=[
            jax.ShapeDtypeStruct((n_rows, _EMBED), jnp.float32),
            jax.ShapeDtypeStruct((n_rows, _EMBED), jnp.float32),
        ],
        scratch_types=[
            pltpu.VMEM((n_chunks, _CHUNK), jnp.int32),
            pltpu.VMEM((_CHUNK, _EMBED), jnp.float32),
            pltpu.VMEM((_CHUNK, _EMBED), jnp.float32),
            pltpu.SemaphoreType.DMA,
            pltpu.SemaphoreType.DMA,
        ],
        mesh=mesh,
    )
    def emb_kernel(src_table, tgt_table, src_idx, tgt_idx,
                   src_out, tgt_out, idx_v, buf0, buf1, gs0, gs1):
        cid = lax.axis_index("c")
        sid = lax.axis_index("s")

        @pl.when(cid == 0)
        def _():
            _gather_side(src_table, src_idx, src_out, idx_v, buf0, buf1,
                         gs0, gs1, sid, n_chunks)

        @pl.when(cid == 1)
        def _():
            _gather_side(tgt_table, tgt_idx, tgt_out, idx_v, buf0, buf1,
                         gs0, gs1, sid, n_chunks)

    return emb_kernel


def kernel(src_table, tgt_table, src_tokens, tgt_tokens):
    b, s = src_tokens.shape
    n_rows = b * s
    n_chunks = n_rows // (_NSUB * _CHUNK)
    src_idx = src_tokens.astype(jnp.int32).reshape(_NSUB, n_chunks, _CHUNK)
    tgt_idx = tgt_tokens.astype(jnp.int32).reshape(_NSUB, n_chunks, _CHUNK)
    src_out, tgt_out = _build(n_rows)(src_table, tgt_table, src_idx, tgt_idx)
    return (
        src_out.reshape(b, s, _EMBED),
        tgt_out.reshape(b, s, _EMBED),
    )


# speedup vs baseline: 1.6854x; 1.4639x over previous
import functools

import jax
import jax.numpy as jnp
from jax import lax
from jax.experimental import pallas as pl
from jax.experimental.pallas import tpu as pltpu
from jax.experimental.pallas import tpu_sc as plsc

_EMBED = 128
_NSUB = 16


def _gather_side(table, idx_hbm, out_hbm, idx_v, buf0, buf1, gs0, gs1,
                 sid, b_per_w):
    base = sid * b_per_w
    pltpu.sync_copy(idx_hbm.at[pl.ds(base, b_per_w)], idx_v)

    pltpu.async_copy(table.at[idx_v.at[0]], buf0, gs0)
    pltpu.async_copy(table.at[idx_v.at[1]], buf1, gs1)

    def body(i, carry):
        j = 2 * i

        pltpu.make_async_copy(table.at[idx_v.at[j]], buf0, gs0).wait()
        pltpu.sync_copy(buf0, out_hbm.at[base + j])

        @pl.when(j + 2 < b_per_w)
        def _():
            pltpu.async_copy(table.at[idx_v.at[j + 2]], buf0, gs0)

        pltpu.make_async_copy(table.at[idx_v.at[j + 1]], buf1, gs1).wait()
        pltpu.sync_copy(buf1, out_hbm.at[base + j + 1])

        @pl.when(j + 3 < b_per_w)
        def _():
            pltpu.async_copy(table.at[idx_v.at[j + 3]], buf1, gs1)

        return carry

    lax.fori_loop(0, b_per_w // 2, body, 0)


@functools.lru_cache(maxsize=None)
def _build(batch, seq):
    assert batch % (_NSUB * 2) == 0
    b_per_w = batch // _NSUB
    mesh = plsc.VectorSubcoreMesh(core_axis_name="c", subcore_axis_name="s")

    @functools.partial(
        pl.kernel,
        out_type=[
            jax.ShapeDtypeStruct((batch, seq, _EMBED), jnp.float32),
            jax.ShapeDtypeStruct((batch, seq, _EMBED), jnp.float32),
        ],
        scratch_types=[
            pltpu.VMEM((b_per_w, seq), jnp.int32),
            pltpu.VMEM((seq, _EMBED), jnp.float32),
            pltpu.VMEM((seq, _EMBED), jnp.float32),
            pltpu.SemaphoreType.DMA,
            pltpu.SemaphoreType.DMA,
        ],
        mesh=mesh,
    )
    def emb_kernel(src_table, tgt_table, src_idx, tgt_idx,
                   src_out, tgt_out, idx_v, buf0, buf1, gs0, gs1):
        cid = lax.axis_index("c")
        sid = lax.axis_index("s")

        @pl.when(cid == 0)
        def _():
            _gather_side(src_table, src_idx, src_out, idx_v, buf0, buf1,
                         gs0, gs1, sid, b_per_w)

        @pl.when(cid == 1)
        def _():
            _gather_side(tgt_table, tgt_idx, tgt_out, idx_v, buf0, buf1,
                         gs0, gs1, sid, b_per_w)

    return emb_kernel


def kernel(src_table, tgt_table, src_tokens, tgt_tokens):
    b, s = src_tokens.shape
    src_idx = src_tokens.astype(jnp.int32)
    tgt_idx = tgt_tokens.astype(jnp.int32)
    return tuple(_build(b, s)(src_table, tgt_table, src_idx, tgt_idx))
